# baseline (speedup 1.0000x reference)
import functools

import jax
import jax.numpy as jnp
from jax import lax
from jax.experimental import pallas as pl
from jax.experimental.pallas import tpu as pltpu
from jax.experimental.pallas import tpu_sc as plsc

_N = 10000
_E = 160000
_WN = 32
_WK = 64
_DEPTH = 4

_NP = 10016
_NPS = _NP // 16
_EP = 245760
_G = _EP // 8
_NW = 32
_RPT = _EP // _NW
_CH = 128
_NCH = _RPT // _CH
_GPT = _G // _NW
_SCH = 16
_NSCH = _GPT // _SCH

_MB = 1024
_MG = _MB // 8

def _sc_gather_body(h_hbm, idx_hbm, out_hbm, idxb, bufs, gsem, ssem):
    wid = lax.axis_index("s") * 2 + lax.axis_index("c")
    base = wid * _RPT
    pltpu.sync_copy(idx_hbm.at[wid], idxb)
    gds = [None] * _NCH
    sds = [None] * _NCH
    for j in range(_NCH):
        b = j % 4
        if j >= 4:
            sds[j - 4].wait()
        gds[j] = pltpu.async_copy(h_hbm.at[idxb.at[j]], bufs.at[b], gsem.at[b])
        if j >= 1:
            k = j - 1
            gds[k].wait()
            sds[k] = pltpu.async_copy(
                bufs.at[k % 4], out_hbm.at[pl.ds(base + k * _CH, _CH)],
                ssem.at[k % 4])
    k = _NCH - 1
    gds[k].wait()
    sds[k] = pltpu.async_copy(
        bufs.at[k % 4], out_hbm.at[pl.ds(base + k * _CH, _CH)], ssem.at[k % 4])
    for k in range(_NCH - 4, _NCH):
        sds[k].wait()


def _sc_scatter_body(aggp_hbm, gnode_hbm, zeros_hbm, out_hbm, rows_v, idx_v,
                     stripe_v, shared, sem):
    cid = lax.axis_index("c")
    sid = lax.axis_index("s")
    wid = sid * 2 + cid
    gbase = wid * _GPT
    pltpu.sync_copy(aggp_hbm.at[pl.ds(gbase, _GPT)], rows_v)
    pltpu.sync_copy(gnode_hbm.at[pl.ds(gbase, _GPT)], idx_v)
    sbase = sid * _NPS
    pltpu.sync_copy(zeros_hbm.at[pl.ds(sbase, _NPS)], stripe_v)
    pltpu.sync_copy(stripe_v, shared.at[pl.ds(sbase, _NPS)])
    plsc.subcore_barrier()
    ds = []
    for j in range(_NSCH):
        idxv = idx_v[pl.ds(j * _SCH, _SCH)]
        ds.append(pltpu.async_copy(
            rows_v.at[pl.ds(j * _SCH, _SCH)], shared.at[idxv], sem, add=True))
    for d in ds:
        d.wait()
    plsc.subcore_barrier()
    pltpu.sync_copy(shared.at[pl.ds(sbase, _NPS)], stripe_v)
    pltpu.sync_copy(stripe_v, out_hbm.at[cid].at[pl.ds(sbase, _NPS)])


@functools.cache
def _sc_kernels():
    mesh = plsc.VectorSubcoreMesh(core_axis_name="c", subcore_axis_name="s",
                                  num_cores=2, num_subcores=16)
    params = pltpu.CompilerParams(use_tc_tiling_on_sc=False)
    gather = pl.kernel(
        _sc_gather_body,
        out_type=jax.ShapeDtypeStruct((_EP, _WN), jnp.float32),
        mesh=mesh,
        compiler_params=params,
        scratch_types=[
            pltpu.VMEM((_NCH, _CH), jnp.int32),
            pltpu.VMEM((4, _CH, _WN), jnp.float32),
            pltpu.SemaphoreType.DMA((4,)),
            pltpu.SemaphoreType.DMA((4,)),
        ],
    )
    scatter = pl.kernel(
        _sc_scatter_body,
        out_type=jax.ShapeDtypeStruct((2, _NP, _WN), jnp.float32),
        mesh=mesh,
        compiler_params=params,
        scratch_types=[
            pltpu.VMEM((_GPT, _WN), jnp.float32),
            pltpu.VMEM((_GPT,), jnp.int32),
            pltpu.VMEM((_NPS, _WN), jnp.float32),
            pltpu.VMEM_SHARED((_NP, _WN), jnp.float32),
            pltpu.SemaphoreType.DMA,
        ],
    )
    return gather, scatter


def _sc_gather(h, idx3):
    return _sc_kernels()[0](h, idx3)


def _sc_scatter(aggp, gnode, zeros_np):
    return _sc_kernels()[1](aggp, gnode, zeros_np)


def _mlp_body(ea_ref, k1w_ref, k1b_ref, k2w_ref, k2b_ref, out_ref):
    e1 = jnp.maximum(
        jnp.dot(ea_ref[...], k1w_ref[...], preferred_element_type=jnp.float32)
        + k1b_ref[...], 0.0)
    out_ref[...] = jnp.maximum(
        jnp.dot(e1, k2w_ref[...], preferred_element_type=jnp.float32)
        + k2b_ref[...], 0.0)


def _main_body(hs_ref, cs_ref, k3r_ref, b3r_ref, out_ref):
    hs = hs_ref[...]
    cs = cs_ref[...]
    csg = cs.reshape(_MG, 8, _WK)
    hsg = hs.reshape(_MG, 8, _WN)
    m = lax.dot_general(csg, hsg, (((1,), (1,)), ((0,), (0,))),
                        preferred_element_type=jnp.float32)
    hsum = jnp.sum(hsg, axis=1)
    out_ref[...] = (
        jnp.dot(m.reshape(_MG, _WK * _WN), k3r_ref[...],
                preferred_element_type=jnp.float32)
        + jnp.dot(hsum, b3r_ref[...], preferred_element_type=jnp.float32))


def _update_body(agg_ref, h_ref, deginv_ref, root_ref, bias_ref, out_ref,
                 *, relu):
    agg = agg_ref[0] + agg_ref[1]
    hn = (agg * deginv_ref[...]
          + jnp.dot(h_ref[...], root_ref[...],
                    preferred_element_type=jnp.float32)
          + bias_ref[...])
    if relu:
        hn = jnp.maximum(hn, 0.0)
    rows = lax.broadcasted_iota(jnp.int32, (_NP, _WN), 0) < _N
    out_ref[...] = jnp.where(rows, hn, 0.0)


def _final_body(h_ref, w2_ref, b2_ref, w3t_ref, b3_ref, out_ref):
    h2 = jnp.maximum(
        jnp.dot(h_ref[...], w2_ref[...], preferred_element_type=jnp.float32)
        + b2_ref[...], 0.0)
    out_ref[...] = (jnp.sum(h2 * w3t_ref[...], axis=1, keepdims=True)
                    + b3_ref[...])


def _mlp(ea_pad, k1_w, k1_b, k2_w, k2_b):
    blk = 2048
    return pl.pallas_call(
        _mlp_body,
        grid=(_EP // blk,),
        in_specs=[
            pl.BlockSpec((blk, 4), lambda i: (i, 0)),
            pl.BlockSpec((4, _WK // 2), lambda i: (0, 0)),
            pl.BlockSpec((_WK // 2,), lambda i: (0,)),
            pl.BlockSpec((_WK // 2, _WK), lambda i: (0, 0)),
            pl.BlockSpec((_WK,), lambda i: (0,)),
        ],
        out_specs=pl.BlockSpec((blk, _WK), lambda i: (i, 0)),
        out_shape=jax.ShapeDtypeStruct((_EP, _WK), jnp.float32),
    )(ea_pad, k1_w, k1_b, k2_w, k2_b)


def _main(hs, cs, k3r, b3r):
    return pl.pallas_call(
        _main_body,
        grid=(_EP // _MB,),
        in_specs=[
            pl.BlockSpec((_MB, _WN), lambda i: (i, 0)),
            pl.BlockSpec((_MB, _WK), lambda i: (i, 0)),
            pl.BlockSpec((_WK * _WN, _WN), lambda i: (0, 0)),
            pl.BlockSpec((_WN, _WN), lambda i: (0, 0)),
        ],
        out_specs=pl.BlockSpec((_MG, _WN), lambda i: (i, 0)),
        out_shape=jax.ShapeDtypeStruct((_G, _WN), jnp.float32),
    )(hs, cs, k3r, b3r)


def _update(agg2, h, deginv, root, bias, relu):
    return pl.pallas_call(
        functools.partial(_update_body, relu=relu),
        in_specs=[
            pl.BlockSpec((2, _NP, _WN), lambda: (0, 0, 0)),
            pl.BlockSpec((_NP, _WN), lambda: (0, 0)),
            pl.BlockSpec((_NP, 1), lambda: (0, 0)),
            pl.BlockSpec((_WN, _WN), lambda: (0, 0)),
            pl.BlockSpec((_WN,), lambda: (0,)),
        ],
        out_specs=pl.BlockSpec((_NP, _WN), lambda: (0, 0)),
        out_shape=jax.ShapeDtypeStruct((_NP, _WN), jnp.float32),
    )(agg2, h, deginv, root, bias)


def _final(h, fc2_w, fc2_b, fc3_w, fc3_b):
    return pl.pallas_call(
        _final_body,
        in_specs=[
            pl.BlockSpec((_NP, _WN), lambda: (0, 0)),
            pl.BlockSpec((_WN, 128), lambda: (0, 0)),
            pl.BlockSpec((128,), lambda: (0,)),
            pl.BlockSpec((1, 128), lambda: (0, 0)),
            pl.BlockSpec((1,), lambda: (0,)),
        ],
        out_specs=pl.BlockSpec((_NP, 1), lambda: (0, 0)),
        out_shape=jax.ShapeDtypeStruct((_NP, 1), jnp.float32),
    )(h, fc2_w, fc2_b, fc3_w.T, fc3_b)


def kernel(x, edge_index, edge_attr, fc1_w, fc1_b, k1_w, k1_b, k2_w, k2_b,
           k3_w, k3_b, root, conv_bias, fc2_w, fc2_b, fc3_w, fc3_b):
    src = edge_index[0]
    dst = edge_index[1]

    order = jnp.argsort(dst).astype(jnp.int32)
    dst_s = jnp.take(dst, order)
    src_s = jnp.take(src, order)
    starts = jnp.searchsorted(
        dst_s, jnp.arange(_N + 1, dtype=jnp.int32)).astype(jnp.int32)
    deg = starts[1:] - starts[:-1]
    gcnt = (deg + 7) // 8
    goff = jnp.concatenate(
        [jnp.zeros((1,), jnp.int32), jnp.cumsum(gcnt).astype(jnp.int32)])
    total_g = goff[_N]
    garange = jnp.arange(_G, dtype=jnp.int32)
    gnode_raw = (jnp.searchsorted(goff, garange, side='right') - 1).astype(jnp.int32)
    gnode_c = jnp.minimum(gnode_raw, _N)
    deg_ext = jnp.concatenate([deg, jnp.zeros((1,), jnp.int32)])
    starts_ext = jnp.concatenate([starts[:_N], jnp.full((1,), _E, jnp.int32)])
    sidx = jnp.arange(_EP, dtype=jnp.int32)
    gs = sidx // 8
    node_s = jnp.take(gnode_c, gs)
    rank = sidx - 8 * jnp.take(goff, node_s)
    valid = rank < jnp.take(deg_ext, node_s)
    epos = jnp.clip(jnp.take(starts_ext, node_s) + rank, 0, _E - 1)
    idx = jnp.where(valid, jnp.take(src_s, epos),
                    _N + (sidx % 16)).astype(jnp.int32)
    eid = jnp.where(valid, jnp.take(order, epos), 0).astype(jnp.int32)
    gnode = jnp.where(garange < total_g, gnode_c,
                      _N + (garange % 16)).astype(jnp.int32)
    deginv = 1.0 / jnp.clip(deg.astype(jnp.float32), 1.0)
    deginv_ext = jnp.concatenate(
        [deginv, jnp.ones((_NP - _N,), jnp.float32)])[:, None]
    idx3 = idx.reshape(_NW, _NCH, _CH)

    ea_pad = jnp.take(edge_attr, eid, axis=0)
    cs = _mlp(ea_pad, k1_w, k1_b, k2_w, k2_b)
    k3r = k3_w.reshape(_WK * _WN, _WN)
    b3r = k3_b.reshape(_WN, _WN)

    h0 = x @ fc1_w + fc1_b
    h = jnp.zeros((_NP, _WN), jnp.float32).at[:_N].set(h0)
    zeros_np = jnp.zeros((_NP, _WN), jnp.float32)

    for d in range(_DEPTH):
        hs = _sc_gather(h, idx3)
        aggp = _main(hs, cs, k3r, b3r)
        agg2 = _sc_scatter(aggp, gnode, zeros_np)
        h = _update(agg2, h, deginv_ext, root, conv_bias, d != _DEPTH - 1)

    out = _final(h, fc2_w, fc2_b, fc3_w, fc3_b)
    return out[:_N]


# speedup vs baseline: 1.0021x; 1.0021x over previous
import functools

import jax
import jax.numpy as jnp
from jax import lax
from jax.experimental import pallas as pl
from jax.experimental.pallas import tpu as pltpu
from jax.experimental.pallas import tpu_sc as plsc

_N = 10000
_E = 160000
_WN = 32
_WK = 64
_DEPTH = 4

_NP = 10016
_NPS = _NP // 16
_EP = 245760
_G = _EP // 8
_NW = 32
_RPT = _EP // _NW
_CH = 128
_NCH = _RPT // _CH
_GPT = _G // _NW
_SCH = 16
_NSCH = _GPT // _SCH

_MB = 1024
_MG = _MB // 8

_RING = 8


def _sc_gather_body(h_hbm, idx_hbm, out_hbm, idxb, bufs, gsem, ssem):
    wid = lax.axis_index("s") * 2 + lax.axis_index("c")
    base = wid * _RPT
    pltpu.sync_copy(idx_hbm.at[wid], idxb)
    gds = [None] * _NCH
    sds = [None] * _NCH
    for j in range(_NCH):
        b = j % _RING
        if j >= _RING:
            sds[j - _RING].wait()
        gds[j] = pltpu.async_copy(h_hbm.at[idxb.at[j]], bufs.at[b], gsem.at[b])
        if j >= _RING - 1:
            k = j - _RING + 1
            gds[k].wait()
            sds[k] = pltpu.async_copy(
                bufs.at[k % _RING], out_hbm.at[pl.ds(base + k * _CH, _CH)],
                ssem.at[k % _RING])
    for k in range(_NCH - _RING + 1, _NCH):
        gds[k].wait()
        sds[k] = pltpu.async_copy(
            bufs.at[k % _RING], out_hbm.at[pl.ds(base + k * _CH, _CH)],
            ssem.at[k % _RING])
    for k in range(_NCH - _RING, _NCH):
        sds[k].wait()


def _sc_scatter_body(aggp_hbm, gnode_hbm, zeros_hbm, out_hbm, rows_v, idx_v,
                     stripe_v, shared, sem):
    cid = lax.axis_index("c")
    sid = lax.axis_index("s")
    wid = sid * 2 + cid
    gbase = wid * _GPT
    pltpu.sync_copy(aggp_hbm.at[pl.ds(gbase, _GPT)], rows_v)
    pltpu.sync_copy(gnode_hbm.at[pl.ds(gbase, _GPT)], idx_v)
    sbase = sid * _NPS
    pltpu.sync_copy(zeros_hbm.at[pl.ds(sbase, _NPS)], stripe_v)
    pltpu.sync_copy(stripe_v, shared.at[pl.ds(sbase, _NPS)])
    plsc.subcore_barrier()
    ds = []
    for j in range(_NSCH):
        idxv = idx_v[pl.ds(j * _SCH, _SCH)]
        ds.append(pltpu.async_copy(
            rows_v.at[pl.ds(j * _SCH, _SCH)], shared.at[idxv], sem, add=True))
    for d in ds:
        d.wait()
    plsc.subcore_barrier()
    pltpu.sync_copy(shared.at[pl.ds(sbase, _NPS)], stripe_v)
    pltpu.sync_copy(stripe_v, out_hbm.at[cid].at[pl.ds(sbase, _NPS)])


@functools.cache
def _sc_kernels():
    mesh = plsc.VectorSubcoreMesh(core_axis_name="c", subcore_axis_name="s",
                                  num_cores=2, num_subcores=16)
    params = pltpu.CompilerParams(use_tc_tiling_on_sc=False)
    gather = pl.kernel(
        _sc_gather_body,
        out_type=jax.ShapeDtypeStruct((_EP, _WN), jnp.float32),
        mesh=mesh,
        compiler_params=params,
        scratch_types=[
            pltpu.VMEM((_NCH, _CH), jnp.int32),
            pltpu.VMEM((_RING, _CH, _WN), jnp.float32),
            pltpu.SemaphoreType.DMA((_RING,)),
            pltpu.SemaphoreType.DMA((_RING,)),
        ],
        name="scgather",
    )
    scatter = pl.kernel(
        _sc_scatter_body,
        out_type=jax.ShapeDtypeStruct((2, _NP, _WN), jnp.float32),
        mesh=mesh,
        compiler_params=params,
        scratch_types=[
            pltpu.VMEM((_GPT, _WN), jnp.float32),
            pltpu.VMEM((_GPT,), jnp.int32),
            pltpu.VMEM((_NPS, _WN), jnp.float32),
            pltpu.VMEM_SHARED((_NP, _WN), jnp.float32),
            pltpu.SemaphoreType.DMA,
        ],
        name="scscatter",
    )
    return gather, scatter


def _sc_gather(h, idx3):
    return _sc_kernels()[0](h, idx3)


def _sc_scatter(aggp, gnode, zeros_np):
    return _sc_kernels()[1](aggp, gnode, zeros_np)


def _mlp_body(ea_ref, k1w_ref, k1b_ref, k2w_ref, k2b_ref, out_ref):
    e1 = jnp.maximum(
        jnp.dot(ea_ref[...], k1w_ref[...], preferred_element_type=jnp.float32)
        + k1b_ref[...], 0.0)
    out_ref[...] = jnp.maximum(
        jnp.dot(e1, k2w_ref[...], preferred_element_type=jnp.float32)
        + k2b_ref[...], 0.0)


def _main_body(hs_ref, cs_ref, k3r_ref, b3r_ref, out_ref):
    hs = hs_ref[...]
    cs = cs_ref[...]
    csg = cs.reshape(_MG, 8, _WK)
    hsg = hs.reshape(_MG, 8, _WN)
    m = lax.dot_general(csg, hsg, (((1,), (1,)), ((0,), (0,))),
                        preferred_element_type=jnp.float32)
    hsum = jnp.sum(hsg, axis=1)
    out_ref[...] = (
        jnp.dot(m.reshape(_MG, _WK * _WN), k3r_ref[...],
                preferred_element_type=jnp.float32)
        + jnp.dot(hsum, b3r_ref[...], preferred_element_type=jnp.float32))


def _update_body(agg_ref, h_ref, deginv_ref, root_ref, bias_ref, out_ref,
                 *, relu):
    agg = agg_ref[0] + agg_ref[1]
    hn = (agg * deginv_ref[...]
          + jnp.dot(h_ref[...], root_ref[...],
                    preferred_element_type=jnp.float32)
          + bias_ref[...])
    if relu:
        hn = jnp.maximum(hn, 0.0)
    rows = lax.broadcasted_iota(jnp.int32, (_NP, _WN), 0) < _N
    out_ref[...] = jnp.where(rows, hn, 0.0)


def _final_body(h_ref, w2_ref, b2_ref, w3t_ref, b3_ref, out_ref):
    h2 = jnp.maximum(
        jnp.dot(h_ref[...], w2_ref[...], preferred_element_type=jnp.float32)
        + b2_ref[...], 0.0)
    out_ref[...] = (jnp.sum(h2 * w3t_ref[...], axis=1, keepdims=True)
                    + b3_ref[...])


def _mlp(ea_pad, k1_w, k1_b, k2_w, k2_b):
    blk = 2048
    return pl.pallas_call(
        _mlp_body,
        grid=(_EP // blk,),
        in_specs=[
            pl.BlockSpec((blk, 4), lambda i: (i, 0)),
            pl.BlockSpec((4, _WK // 2), lambda i: (0, 0)),
            pl.BlockSpec((_WK // 2,), lambda i: (0,)),
            pl.BlockSpec((_WK // 2, _WK), lambda i: (0, 0)),
            pl.BlockSpec((_WK,), lambda i: (0,)),
        ],
        out_specs=pl.BlockSpec((blk, _WK), lambda i: (i, 0)),
        out_shape=jax.ShapeDtypeStruct((_EP, _WK), jnp.float32),
        name="tcmlp",
    )(ea_pad, k1_w, k1_b, k2_w, k2_b)


def _main(hs, cs, k3r, b3r):
    return pl.pallas_call(
        _main_body,
        grid=(_EP // _MB,),
        in_specs=[
            pl.BlockSpec((_MB, _WN), lambda i: (i, 0)),
            pl.BlockSpec((_MB, _WK), lambda i: (i, 0)),
            pl.BlockSpec((_WK * _WN, _WN), lambda i: (0, 0)),
            pl.BlockSpec((_WN, _WN), lambda i: (0, 0)),
        ],
        out_specs=pl.BlockSpec((_MG, _WN), lambda i: (i, 0)),
        out_shape=jax.ShapeDtypeStruct((_G, _WN), jnp.float32),
        name="tcmain",
    )(hs, cs, k3r, b3r)


def _update(agg2, h, deginv, root, bias, relu):
    return pl.pallas_call(
        functools.partial(_update_body, relu=relu),
        in_specs=[
            pl.BlockSpec((2, _NP, _WN), lambda: (0, 0, 0)),
            pl.BlockSpec((_NP, _WN), lambda: (0, 0)),
            pl.BlockSpec((_NP, 1), lambda: (0, 0)),
            pl.BlockSpec((_WN, _WN), lambda: (0, 0)),
            pl.BlockSpec((_WN,), lambda: (0,)),
        ],
        out_specs=pl.BlockSpec((_NP, _WN), lambda: (0, 0)),
        out_shape=jax.ShapeDtypeStruct((_NP, _WN), jnp.float32),
        name="tcupdate",
    )(agg2, h, deginv, root, bias)


def _final(h, fc2_w, fc2_b, fc3_w, fc3_b):
    return pl.pallas_call(
        _final_body,
        in_specs=[
            pl.BlockSpec((_NP, _WN), lambda: (0, 0)),
            pl.BlockSpec((_WN, 128), lambda: (0, 0)),
            pl.BlockSpec((128,), lambda: (0,)),
            pl.BlockSpec((1, 128), lambda: (0, 0)),
            pl.BlockSpec((1,), lambda: (0,)),
        ],
        out_specs=pl.BlockSpec((_NP, 1), lambda: (0, 0)),
        out_shape=jax.ShapeDtypeStruct((_NP, 1), jnp.float32),
        name="tcfinal",
    )(h, fc2_w, fc2_b, fc3_w.T, fc3_b)


def kernel(x, edge_index, edge_attr, fc1_w, fc1_b, k1_w, k1_b, k2_w, k2_b,
           k3_w, k3_b, root, conv_bias, fc2_w, fc2_b, fc3_w, fc3_b):
    src = edge_index[0]
    dst = edge_index[1]

    order = jnp.argsort(dst).astype(jnp.int32)
    dst_s = jnp.take(dst, order)
    src_s = jnp.take(src, order)
    starts = jnp.searchsorted(
        dst_s, jnp.arange(_N + 1, dtype=jnp.int32)).astype(jnp.int32)
    deg = starts[1:] - starts[:-1]
    gcnt = (deg + 7) // 8
    goff = jnp.concatenate(
        [jnp.zeros((1,), jnp.int32), jnp.cumsum(gcnt).astype(jnp.int32)])
    total_g = goff[_N]
    garange = jnp.arange(_G, dtype=jnp.int32)
    gnode_raw = (jnp.searchsorted(goff, garange, side='right') - 1).astype(jnp.int32)
    gnode_c = jnp.minimum(gnode_raw, _N)
    deg_ext = jnp.concatenate([deg, jnp.zeros((1,), jnp.int32)])
    starts_ext = jnp.concatenate([starts[:_N], jnp.full((1,), _E, jnp.int32)])
    sidx = jnp.arange(_EP, dtype=jnp.int32)
    gs = sidx // 8
    node_s = jnp.take(gnode_c, gs)
    rank = sidx - 8 * jnp.take(goff, node_s)
    valid = rank < jnp.take(deg_ext, node_s)
    epos = jnp.clip(jnp.take(starts_ext, node_s) + rank, 0, _E - 1)
    idx = jnp.where(valid, jnp.take(src_s, epos),
                    _N + (sidx % 16)).astype(jnp.int32)
    eid = jnp.where(valid, jnp.take(order, epos), 0).astype(jnp.int32)
    gnode = jnp.where(garange < total_g, gnode_c,
                      _N + (garange % 16)).astype(jnp.int32)
    deginv = 1.0 / jnp.clip(deg.astype(jnp.float32), 1.0)
    deginv_ext = jnp.concatenate(
        [deginv, jnp.ones((_NP - _N,), jnp.float32)])[:, None]
    idx3 = idx.reshape(_NW, _NCH, _CH)

    ea_pad = jnp.take(edge_attr, eid, axis=0)
    cs = _mlp(ea_pad, k1_w, k1_b, k2_w, k2_b)
    k3r = k3_w.reshape(_WK * _WN, _WN)
    b3r = k3_b.reshape(_WN, _WN)

    h0 = x @ fc1_w + fc1_b
    h = jnp.zeros((_NP, _WN), jnp.float32).at[:_N].set(h0)
    zeros_np = jnp.zeros((_NP, _WN), jnp.float32)

    for d in range(_DEPTH):
        hs = _sc_gather(h, idx3)
        aggp = _main(hs, cs, k3r, b3r)
        agg2 = _sc_scatter(aggp, gnode, zeros_np)
        h = _update(agg2, h, deginv_ext, root, conv_bias, d != _DEPTH - 1)

    out = _final(h, fc2_w, fc2_b, fc3_w, fc3_b)
    return out[:_N]
